# baseline (speedup 1.0000x reference)
import functools

import jax
import jax.numpy as jnp
from jax import lax
from jax.experimental import pallas as pl
from jax.experimental.pallas import tpu as pltpu
from jax.experimental.pallas import tpu_sc as plsc

_NUM_EMB = 1024
_DIM = 256
_TILE = 256
_TOKENS = 32 * 576
_STEPS = _TOKENS // _TILE

_NC = 2
_NS = 16
_NW = _NC * _NS
_B_PER_W = _TOKENS // _NW
_CHUNK = 288


def _vq_tc_body(x_ref, e_ref, idx_ref, counts_ref, loss_ref, perp_ref):
    i = pl.program_id(0)
    nsteps = pl.num_programs(0)

    @pl.when(i == 0)
    def _init():
        counts_ref[...] = jnp.zeros_like(counts_ref)
        loss_ref[0, 0] = jnp.float32(0.0)

    x = x_ref[...]
    e = e_ref[...]
    x2 = jnp.sum(x * x, axis=1)
    e2 = jnp.sum(e * e, axis=1)
    mm = lax.dot_general(x, e, (((1,), (1,)), ((), ())),
                         preferred_element_type=jnp.float32)
    d = (x2[:, None] + e2[None, :]) - 2.0 * mm
    row_min = jnp.min(d, axis=1)
    col = lax.broadcasted_iota(jnp.int32, d.shape, 1)
    idx = jnp.min(jnp.where(d == row_min[:, None], col, _NUM_EMB), axis=1)
    idx_ref[0, 0, :] = idx
    onehot = (col == idx[:, None]).astype(jnp.float32)
    counts_ref[...] += jnp.sum(onehot, axis=0, keepdims=True)
    loss_ref[0, 0] += jnp.sum(row_min)

    @pl.when(i == nsteps - 1)
    def _fini():
        v = loss_ref[0, 0] / jnp.float32(_TOKENS * _DIM)
        loss_ref[0, 0] = v + 0.25 * v
        p = counts_ref[...] / jnp.float32(_TOKENS)
        ent = -jnp.sum(p * jnp.log(p + 1e-10))
        perp_ref[0, 0] = jnp.exp(ent)


_vq_tc = pl.pallas_call(
    _vq_tc_body,
    grid=(_STEPS,),
    in_specs=[
        pl.BlockSpec((_TILE, _DIM), lambda i: (i, 0)),
        pl.BlockSpec((_NUM_EMB, _DIM), lambda i: (0, 0)),
    ],
    out_specs=[
        pl.BlockSpec((1, 1, _TILE), lambda i: (i, 0, 0)),
        pl.BlockSpec((1, _NUM_EMB), lambda i: (0, 0)),
        pl.BlockSpec(memory_space=pltpu.SMEM),
        pl.BlockSpec(memory_space=pltpu.SMEM),
    ],
    out_shape=[
        jax.ShapeDtypeStruct((_STEPS, 1, _TILE), jnp.int32),
        jax.ShapeDtypeStruct((1, _NUM_EMB), jnp.float32),
        jax.ShapeDtypeStruct((1, 1), jnp.float32),
        jax.ShapeDtypeStruct((1, 1), jnp.float32),
    ],
)


@functools.cache
def _make_sc_gather():
    @functools.partial(
        pl.kernel,
        mesh=plsc.VectorSubcoreMesh(core_axis_name="c", subcore_axis_name="s"),
        out_type=jax.ShapeDtypeStruct((_TOKENS, _DIM), jnp.float32),
        scratch_types=[
            pltpu.VMEM((_CHUNK,), jnp.int32),
            pltpu.VMEM((_CHUNK, _DIM), jnp.float32),
            pltpu.SemaphoreType.DMA,
        ],
    )
    def _sc_gather(table_hbm, idx_hbm, out_hbm, idx_v, rows_v, sem):
        wid = lax.axis_index("s") * _NC + lax.axis_index("c")
        base = wid * _B_PER_W
        for c in range(_B_PER_W // _CHUNK):
            off = base + c * _CHUNK
            pltpu.sync_copy(idx_hbm.at[pl.ds(off, _CHUNK)], idx_v)
            pltpu.async_copy(table_hbm.at[idx_v], rows_v, sem).wait()
            pltpu.sync_copy(rows_v, out_hbm.at[pl.ds(off, _CHUNK)])

    return _sc_gather


def kernel(inputs, embedding_weight):
    batch, seq, dim = inputs.shape
    flat = inputs.reshape(-1, dim)
    idx3, counts, loss, perp = _vq_tc(flat, embedding_weight)
    idx = idx3.reshape(-1)
    quantized = _make_sc_gather()(embedding_weight, idx)
    return (quantized.reshape(batch, seq, dim),
            loss.reshape(()),
            perp.reshape(()),
            idx.reshape(batch, seq))


# speedup vs baseline: 1.0278x; 1.0278x over previous
import functools

import jax
import jax.numpy as jnp
from jax import lax
from jax.experimental import pallas as pl
from jax.experimental.pallas import tpu as pltpu
from jax.experimental.pallas import tpu_sc as plsc

_NUM_EMB = 1024
_DIM = 256
_TILE = 256
_TOKENS = 32 * 576
_STEPS = _TOKENS // _TILE

_NC = 2
_NS = 16
_NW = _NC * _NS
_B_PER_W = _TOKENS // _NW
_CHUNK = 288


def _e2_body(e_ref, e2_ref):
    e = e_ref[...]
    e2_ref[...] = jnp.sum(e * e, axis=1)[None, :]


_e2_pre = pl.pallas_call(
    _e2_body,
    out_shape=jax.ShapeDtypeStruct((1, _NUM_EMB), jnp.float32),
)


def _vq_tc_body(x_ref, e_ref, e2_ref, idx_ref, counts_ref, loss_ref, perp_ref):
    i = pl.program_id(0)
    nsteps = pl.num_programs(0)

    @pl.when(i == 0)
    def _init():
        counts_ref[...] = jnp.zeros_like(counts_ref)
        loss_ref[0, 0] = jnp.float32(0.0)

    x = x_ref[...]
    e = e_ref[...]
    x2 = jnp.sum(x * x, axis=1)
    e2 = e2_ref[0, :]
    mm = lax.dot_general(x, e, (((1,), (1,)), ((), ())),
                         preferred_element_type=jnp.float32)
    d = (x2[:, None] + e2[None, :]) - 2.0 * mm
    row_min = jnp.min(d, axis=1)
    col = lax.broadcasted_iota(jnp.int32, d.shape, 1)
    idx = jnp.min(jnp.where(d == row_min[:, None], col, _NUM_EMB), axis=1)
    idx_ref[0, 0, :] = idx
    onehot = (col == idx[:, None]).astype(jnp.float32)
    ones_row = jnp.ones((1, _TILE), jnp.float32)
    counts_ref[...] += lax.dot_general(
        ones_row, onehot, (((1,), (0,)), ((), ())),
        preferred_element_type=jnp.float32)
    loss_ref[0, 0] += jnp.sum(row_min)

    @pl.when(i == nsteps - 1)
    def _fini():
        v = loss_ref[0, 0] / jnp.float32(_TOKENS * _DIM)
        loss_ref[0, 0] = v + 0.25 * v
        p = counts_ref[...] / jnp.float32(_TOKENS)
        ent = -jnp.sum(p * jnp.log(p + 1e-10))
        perp_ref[0, 0] = jnp.exp(ent)


_vq_tc = pl.pallas_call(
    _vq_tc_body,
    grid=(_STEPS,),
    in_specs=[
        pl.BlockSpec((_TILE, _DIM), lambda i: (i, 0)),
        pl.BlockSpec((_NUM_EMB, _DIM), lambda i: (0, 0)),
        pl.BlockSpec((1, _NUM_EMB), lambda i: (0, 0)),
    ],
    out_specs=[
        pl.BlockSpec((1, 1, _TILE), lambda i: (i, 0, 0)),
        pl.BlockSpec((1, _NUM_EMB), lambda i: (0, 0)),
        pl.BlockSpec(memory_space=pltpu.SMEM),
        pl.BlockSpec(memory_space=pltpu.SMEM),
    ],
    out_shape=[
        jax.ShapeDtypeStruct((_STEPS, 1, _TILE), jnp.int32),
        jax.ShapeDtypeStruct((1, _NUM_EMB), jnp.float32),
        jax.ShapeDtypeStruct((1, 1), jnp.float32),
        jax.ShapeDtypeStruct((1, 1), jnp.float32),
    ],
)


@functools.cache
def _make_sc_gather():
    @functools.partial(
        pl.kernel,
        mesh=plsc.VectorSubcoreMesh(core_axis_name="c", subcore_axis_name="s"),
        out_type=jax.ShapeDtypeStruct((_TOKENS, _DIM), jnp.float32),
        scratch_types=[
            pltpu.VMEM((_CHUNK,), jnp.int32),
            pltpu.VMEM((_CHUNK, _DIM), jnp.float32),
            pltpu.SemaphoreType.DMA,
        ],
    )
    def _sc_gather(table_hbm, idx_hbm, out_hbm, idx_v, rows_v, sem):
        wid = lax.axis_index("s") * _NC + lax.axis_index("c")
        base = wid * _B_PER_W
        for c in range(_B_PER_W // _CHUNK):
            off = base + c * _CHUNK
            pltpu.sync_copy(idx_hbm.at[pl.ds(off, _CHUNK)], idx_v)
            pltpu.async_copy(table_hbm.at[idx_v], rows_v, sem).wait()
            pltpu.sync_copy(rows_v, out_hbm.at[pl.ds(off, _CHUNK)])

    return _sc_gather


def kernel(inputs, embedding_weight):
    batch, seq, dim = inputs.shape
    flat = inputs.reshape(-1, dim)
    e2 = _e2_pre(embedding_weight)
    idx3, counts, loss, perp = _vq_tc(flat, embedding_weight, e2)
    idx = idx3.reshape(-1)
    quantized = _make_sc_gather()(embedding_weight, idx)
    return (quantized.reshape(batch, seq, dim),
            loss.reshape(()),
            perp.reshape(()),
            idx.reshape(batch, seq))
